# baseline (speedup 1.0000x reference)
import functools

import jax
import jax.numpy as jnp
from jax.experimental import pallas as pl

_K = 32
_ROWS = 64


_SUB = 64
_LANES = 128
_PER_COL = 8


def _cmpex_sub(v, ix, j, k):
    rows, s, lanes = v.shape
    g = s // (2 * j)
    v5 = v.reshape(rows, g, 2, j, lanes)
    i5 = ix.reshape(rows, g, 2, j, lanes)
    av, bv = v5[:, :, 0], v5[:, :, 1]
    ai, bi = i5[:, :, 0], i5[:, :, 1]
    le = av <= bv
    if k is None:
        keep = le
    else:
        q = jax.lax.broadcasted_iota(jnp.int32, (1, g, 1, 1), 1)
        asc = ((q * (2 * j)) & k) == 0
        keep = le == asc
    lo_v = jnp.where(keep, av, bv)
    hi_v = jnp.where(keep, bv, av)
    lo_i = jnp.where(keep, ai, bi)
    hi_i = jnp.where(keep, bi, ai)
    v = jnp.concatenate([lo_v[:, :, None], hi_v[:, :, None]], axis=2)
    ix = jnp.concatenate([lo_i[:, :, None], hi_i[:, :, None]], axis=2)
    return v.reshape(rows, s, lanes), ix.reshape(rows, s, lanes)


def _knn_body(t_ref, s_ref, idx_ref, val_ref):
    t = t_ref[0]
    s = s_ref[0]
    mm = jax.lax.dot_general(
        t, s, (((1,), (1,)), ((), ())), preferred_element_type=jnp.float32
    )
    r0 = jnp.sum(t * t, axis=1, keepdims=True)
    r1 = jnp.sum(s * s, axis=1)[None, :]
    d = (r0 - 2.0 * mm) + r1

    rows, ns = d.shape
    inf = jnp.float32(jnp.inf)

    v = d.reshape(rows, _SUB, _LANES)
    ix = (
        jax.lax.broadcasted_iota(jnp.int32, (rows, _SUB, _LANES), 1) * _LANES
        + jax.lax.broadcasted_iota(jnp.int32, (rows, _SUB, _LANES), 2)
    )
    for k in (2, 4, 8):
        j = k // 2
        while j:
            v, ix = _cmpex_sub(v, ix, j, k)
            j //= 2
    for _ in range(3):
        s_half = v.shape[1] // 2
        v4 = v.reshape(rows, s_half // 8, 16, _LANES)
        i4 = ix.reshape(rows, s_half // 8, 16, _LANES)
        av, bv = v4[:, :, :8], v4[:, :, 8:]
        ai, bi = i4[:, :, :8], i4[:, :, 8:]
        le = av <= bv
        v = jnp.where(le, av, bv).reshape(rows, s_half, _LANES)
        ix = jnp.where(le, ai, bi).reshape(rows, s_half, _LANES)
        last = s_half == 8
        for j in (4, 2, 1):
            v, ix = _cmpex_sub(v, ix, j, None if last else 8)
    cv = v
    gi = ix

    big_i = jnp.int32(ns)
    vals, idxs = [], []
    for _ in range(_K):
        m = jnp.min(cv, axis=(1, 2), keepdims=True)
        eq = cv == m
        im = jnp.min(jnp.where(eq, gi, big_i), axis=(1, 2), keepdims=True)
        vals.append(m[:, 0])
        idxs.append(im[:, 0])
        cv = jnp.where(gi == im, inf, cv)
    val_ref[0] = jnp.concatenate(vals, axis=1)
    idx_ref[0] = jnp.concatenate(idxs, axis=1)


@functools.partial(jax.jit, static_argnames=("interpret",))
def _impl(source, target, interpret=False):
    b, nt, _ = target.shape
    ns = source.shape[1]
    grid = (b, nt // _ROWS)
    idx, vals = pl.pallas_call(
        _knn_body,
        grid=grid,
        in_specs=[
            pl.BlockSpec((1, _ROWS, 3), lambda bi, i: (bi, i, 0)),
            pl.BlockSpec((1, ns, 3), lambda bi, i: (bi, 0, 0)),
        ],
        out_specs=[
            pl.BlockSpec((1, _ROWS, _K), lambda bi, i: (bi, i, 0)),
            pl.BlockSpec((1, _ROWS, _K), lambda bi, i: (bi, i, 0)),
        ],
        out_shape=[
            jax.ShapeDtypeStruct((b, nt, _K), jnp.int32),
            jax.ShapeDtypeStruct((b, nt, _K), jnp.float32),
        ],
        interpret=interpret,
    )(target, source)
    batch_idx = jnp.broadcast_to(
        jnp.arange(b, dtype=jnp.int32).reshape(b, 1, 1), (b, nt, _K)
    )
    patches_idx = jnp.stack([batch_idx, idx], axis=-1)
    return patches_idx, vals


def kernel(source, target):
    return _impl(source, target)


# speedup vs baseline: 2.5534x; 2.5534x over previous
import functools

import jax
import jax.numpy as jnp
from jax.experimental import pallas as pl

_K = 32
_ROWS = 256


_SUB = 64
_LANES = 128
_PER_COL = 7


def _knn_body(t_ref, s_ref, idx_ref, val_ref):
    t = t_ref[0]
    s = s_ref[0]
    mm = jax.lax.dot_general(
        t, s, (((1,), (1,)), ((), ())), preferred_element_type=jnp.float32
    )
    r0 = jnp.sum(t * t, axis=1, keepdims=True)
    r1 = jnp.sum(s * s, axis=1)[None, :]
    d = (r0 - 2.0 * mm) + r1

    rows, ns = d.shape
    inf = jnp.float32(jnp.inf)

    v = d.reshape(rows, _SUB, _LANES)
    sub = jax.lax.broadcasted_iota(jnp.int32, (rows, _SUB, _LANES), 1)
    big_sub = jnp.int32(_SUB)
    c_vals, c_sub = [], []
    for _ in range(_PER_COL):
        m = jnp.min(v, axis=1, keepdims=True)
        eq = v == m
        im = jnp.min(jnp.where(eq, sub, big_sub), axis=1, keepdims=True)
        c_vals.append(m)
        c_sub.append(im)
        v = jnp.where(sub == im, inf, v)
    lane = jax.lax.broadcasted_iota(jnp.int32, (rows, 1, _LANES), 2)
    pv = c_vals
    pg = [c * _LANES + lane for c in c_sub]

    big_i = jnp.int32(ns)
    vals, idxs = [], []
    for i in range(_K):
        depth = min(i + 1, _PER_COL)
        mv, mg = pv[0], pg[0]
        for k in range(1, depth):
            le = mv <= pv[k]
            mv = jnp.where(le, mv, pv[k])
            mg = jnp.where(le, mg, pg[k])
        m = jnp.min(mv, axis=2, keepdims=True)
        im = jnp.min(jnp.where(mv == m, mg, big_i), axis=2, keepdims=True)
        vals.append(m[:, 0])
        idxs.append(im[:, 0])
        for k in range(depth):
            pv[k] = jnp.where(pg[k] == im, inf, pv[k])
    val_ref[0] = jnp.concatenate(vals, axis=1)
    idx_ref[0] = jnp.concatenate(idxs, axis=1)


@functools.partial(jax.jit, static_argnames=("interpret",))
def _impl(source, target, interpret=False):
    b, nt, _ = target.shape
    ns = source.shape[1]
    grid = (b, nt // _ROWS)
    idx, vals = pl.pallas_call(
        _knn_body,
        grid=grid,
        in_specs=[
            pl.BlockSpec((1, _ROWS, 3), lambda bi, i: (bi, i, 0)),
            pl.BlockSpec((1, ns, 3), lambda bi, i: (bi, 0, 0)),
        ],
        out_specs=[
            pl.BlockSpec((1, _ROWS, _K), lambda bi, i: (bi, i, 0)),
            pl.BlockSpec((1, _ROWS, _K), lambda bi, i: (bi, i, 0)),
        ],
        out_shape=[
            jax.ShapeDtypeStruct((b, nt, _K), jnp.int32),
            jax.ShapeDtypeStruct((b, nt, _K), jnp.float32),
        ],
        interpret=interpret,
    )(target, source)
    batch_idx = jnp.broadcast_to(
        jnp.arange(b, dtype=jnp.int32).reshape(b, 1, 1), (b, nt, _K)
    )
    patches_idx = jnp.stack([batch_idx, idx], axis=-1)
    return patches_idx, vals


def kernel(source, target):
    return _impl(source, target)


# speedup vs baseline: 3.6746x; 1.4391x over previous
import functools

import jax
import jax.numpy as jnp
from jax.experimental import pallas as pl

_K = 32
_ROWS = 256


_SUB = 64
_LANES = 128
_PER_COL = 7


def _knn_body(t_ref, s_ref, idx_ref, val_ref):
    t = t_ref[0]
    s = s_ref[0]
    mm = jax.lax.dot_general(
        t, s, (((1,), (1,)), ((), ())), preferred_element_type=jnp.float32
    )
    r0 = jnp.sum(t * t, axis=1, keepdims=True)
    r1 = jnp.sum(s * s, axis=1)[None, :]
    d = (r0 - 2.0 * mm) + r1

    rows, ns = d.shape
    inf = jnp.float32(jnp.inf)

    v = d.reshape(rows, _SUB, _LANES)
    sub = jax.lax.broadcasted_iota(jnp.int32, (rows, _SUB, _LANES), 1)
    big_sub = jnp.int32(_SUB)
    c_vals, c_sub = [], []
    for _ in range(_PER_COL):
        m = jnp.min(v, axis=1, keepdims=True)
        eq = v == m
        im = jnp.min(jnp.where(eq, sub, big_sub), axis=1, keepdims=True)
        c_vals.append(m)
        c_sub.append(im)
        v = jnp.where(sub == im, inf, v)
    cv = jnp.concatenate(c_vals, axis=1)
    ci = jnp.concatenate(c_sub, axis=1)
    lane = jax.lax.broadcasted_iota(jnp.int32, (rows, _PER_COL, _LANES), 2)
    gi = ci * _LANES + lane

    big_i = jnp.int32(ns)
    vals, idxs = [], []
    for _ in range(_K):
        m = jnp.min(cv, axis=(1, 2), keepdims=True)
        eq = cv == m
        im = jnp.min(jnp.where(eq, gi, big_i), axis=(1, 2), keepdims=True)
        vals.append(m[:, 0])
        idxs.append(im[:, 0])
        cv = jnp.where(gi == im, inf, cv)
    val_ref[0] = jnp.concatenate(vals, axis=1)
    idx_ref[0] = jnp.concatenate(idxs, axis=1)


@functools.partial(jax.jit, static_argnames=("interpret",))
def _impl(source, target, interpret=False):
    b, nt, _ = target.shape
    ns = source.shape[1]
    grid = (b, nt // _ROWS)
    idx, vals = pl.pallas_call(
        _knn_body,
        grid=grid,
        in_specs=[
            pl.BlockSpec((1, _ROWS, 3), lambda bi, i: (bi, i, 0)),
            pl.BlockSpec((1, ns, 3), lambda bi, i: (bi, 0, 0)),
        ],
        out_specs=[
            pl.BlockSpec((1, _ROWS, _K), lambda bi, i: (bi, i, 0)),
            pl.BlockSpec((1, _ROWS, _K), lambda bi, i: (bi, i, 0)),
        ],
        out_shape=[
            jax.ShapeDtypeStruct((b, nt, _K), jnp.int32),
            jax.ShapeDtypeStruct((b, nt, _K), jnp.float32),
        ],
        interpret=interpret,
    )(target, source)
    batch_idx = jnp.broadcast_to(
        jnp.arange(b, dtype=jnp.int32).reshape(b, 1, 1), (b, nt, _K)
    )
    patches_idx = jnp.stack([batch_idx, idx], axis=-1)
    return patches_idx, vals


def kernel(source, target):
    return _impl(source, target)


# speedup vs baseline: 3.7128x; 1.0104x over previous
import functools

import jax
import jax.numpy as jnp
from jax.experimental import pallas as pl

_K = 32
_ROWS = 256


_SUB = 64
_LANES = 128
_PER_COL = 8


def _knn_body(t_ref, s_ref, idx_ref, val_ref):
    t = t_ref[0]
    s = s_ref[0]
    mm = jax.lax.dot_general(
        t, s, (((1,), (1,)), ((), ())), preferred_element_type=jnp.float32
    )
    r0 = jnp.sum(t * t, axis=1, keepdims=True)
    r1 = jnp.sum(s * s, axis=1)[None, :]
    d = (r0 - 2.0 * mm) + r1

    rows, ns = d.shape
    inf = jnp.float32(jnp.inf)

    v = d.reshape(rows, _SUB, _LANES)
    sub = jax.lax.broadcasted_iota(jnp.int32, (rows, _SUB, _LANES), 1)
    big_sub = jnp.int32(_SUB)
    c_vals, c_sub = [], []
    for _ in range(_PER_COL):
        m = jnp.min(v, axis=1, keepdims=True)
        eq = v == m
        im = jnp.min(jnp.where(eq, sub, big_sub), axis=1, keepdims=True)
        c_vals.append(m)
        c_sub.append(im)
        v = jnp.where(sub == im, inf, v)
    cv = jnp.concatenate(c_vals, axis=1)
    ci = jnp.concatenate(c_sub, axis=1)
    lane = jax.lax.broadcasted_iota(jnp.int32, (rows, _PER_COL, _LANES), 2)
    gi = ci * _LANES + lane

    big_i = jnp.int32(ns)
    vals, idxs = [], []
    for _ in range(_K):
        m = jnp.min(cv, axis=(1, 2), keepdims=True)
        eq = cv == m
        im = jnp.min(jnp.where(eq, gi, big_i), axis=(1, 2), keepdims=True)
        vals.append(m[:, 0])
        idxs.append(im[:, 0])
        cv = jnp.where(gi == im, inf, cv)
    val_ref[0] = jnp.concatenate(vals, axis=1)
    idx_ref[0] = jnp.concatenate(idxs, axis=1)


@functools.partial(jax.jit, static_argnames=("interpret",))
def _impl(source, target, interpret=False):
    b, nt, _ = target.shape
    ns = source.shape[1]
    grid = (b, nt // _ROWS)
    idx, vals = pl.pallas_call(
        _knn_body,
        grid=grid,
        in_specs=[
            pl.BlockSpec((1, _ROWS, 3), lambda bi, i: (bi, i, 0)),
            pl.BlockSpec((1, ns, 3), lambda bi, i: (bi, 0, 0)),
        ],
        out_specs=[
            pl.BlockSpec((1, _ROWS, _K), lambda bi, i: (bi, i, 0)),
            pl.BlockSpec((1, _ROWS, _K), lambda bi, i: (bi, i, 0)),
        ],
        out_shape=[
            jax.ShapeDtypeStruct((b, nt, _K), jnp.int32),
            jax.ShapeDtypeStruct((b, nt, _K), jnp.float32),
        ],
        interpret=interpret,
    )(target, source)
    batch_idx = jnp.broadcast_to(
        jnp.arange(b, dtype=jnp.int32).reshape(b, 1, 1), (b, nt, _K)
    )
    patches_idx = jnp.stack([batch_idx, idx], axis=-1)
    return patches_idx, vals


def kernel(source, target):
    return _impl(source, target)


# speedup vs baseline: 4.3984x; 1.1847x over previous
import functools

import jax
import jax.numpy as jnp
from jax.experimental import pallas as pl

_K = 32
_ROWS = 128


_SUB = 64
_LANES = 128
_PER_COL = 8


def _knn_body(t_ref, s_ref, idx_ref, val_ref):
    t = t_ref[0]
    s = s_ref[0]
    mm = jax.lax.dot_general(
        t, s, (((1,), (1,)), ((), ())), preferred_element_type=jnp.float32
    )
    r0 = jnp.sum(t * t, axis=1, keepdims=True)
    r1 = jnp.sum(s * s, axis=1)[None, :]
    d = (r0 - 2.0 * mm) + r1

    rows, ns = d.shape
    inf = jnp.float32(jnp.inf)

    v = d.reshape(rows, _SUB, _LANES)
    sub = jax.lax.broadcasted_iota(jnp.int32, (rows, _SUB, _LANES), 1)
    big_sub = jnp.int32(_SUB)
    c_vals, c_sub = [], []
    for _ in range(_PER_COL):
        m = jnp.min(v, axis=1, keepdims=True)
        eq = v == m
        im = jnp.min(jnp.where(eq, sub, big_sub), axis=1, keepdims=True)
        c_vals.append(m)
        c_sub.append(im)
        v = jnp.where(sub == im, inf, v)
    cv = jnp.concatenate(c_vals, axis=1)
    ci = jnp.concatenate(c_sub, axis=1)
    lane = jax.lax.broadcasted_iota(jnp.int32, (rows, _PER_COL, _LANES), 2)
    gi = ci * _LANES + lane

    big_i = jnp.int32(ns)
    vals, idxs = [], []
    for _ in range(_K):
        m = jnp.min(cv, axis=(1, 2), keepdims=True)
        eq = cv == m
        im = jnp.min(jnp.where(eq, gi, big_i), axis=(1, 2), keepdims=True)
        vals.append(m[:, 0])
        idxs.append(im[:, 0])
        cv = jnp.where(gi == im, inf, cv)
    val_ref[0] = jnp.concatenate(vals, axis=1)
    idx_ref[0] = jnp.concatenate(idxs, axis=1)


@functools.partial(jax.jit, static_argnames=("interpret",))
def _impl(source, target, interpret=False):
    b, nt, _ = target.shape
    ns = source.shape[1]
    grid = (b, nt // _ROWS)
    idx, vals = pl.pallas_call(
        _knn_body,
        grid=grid,
        in_specs=[
            pl.BlockSpec((1, _ROWS, 3), lambda bi, i: (bi, i, 0)),
            pl.BlockSpec((1, ns, 3), lambda bi, i: (bi, 0, 0)),
        ],
        out_specs=[
            pl.BlockSpec((1, _ROWS, _K), lambda bi, i: (bi, i, 0)),
            pl.BlockSpec((1, _ROWS, _K), lambda bi, i: (bi, i, 0)),
        ],
        out_shape=[
            jax.ShapeDtypeStruct((b, nt, _K), jnp.int32),
            jax.ShapeDtypeStruct((b, nt, _K), jnp.float32),
        ],
        interpret=interpret,
    )(target, source)
    batch_idx = jnp.broadcast_to(
        jnp.arange(b, dtype=jnp.int32).reshape(b, 1, 1), (b, nt, _K)
    )
    patches_idx = jnp.stack([batch_idx, idx], axis=-1)
    return patches_idx, vals


def kernel(source, target):
    return _impl(source, target)
